# initial kernel scaffold (unmeasured)
import jax
import jax.numpy as jnp
from jax import lax
from jax.experimental import pallas as pl
from jax.experimental.pallas import tpu as pltpu


def kernel(
    x,
):
    def body(*refs):
        pass

    out_shape = jax.ShapeDtypeStruct(..., jnp.float32)
    return pl.pallas_call(body, out_shape=out_shape)(...)


# baseline (device time: 342000 ns/iter reference)
import jax
import jax.numpy as jnp
from jax import lax
from jax.experimental import pallas as pl
from jax.experimental.pallas import tpu as pltpu

Z = 4


def kernel(x):
    m, n = x.shape
    blk = n // Z
    x = x.astype(jnp.bfloat16)

    def body(x_ref, out_ref, send_sems, recv_sems, local_sem):
        my_x = lax.axis_index("x")
        my_y = lax.axis_index("y")
        my_z = lax.axis_index("z")

        barrier = pltpu.get_barrier_semaphore()
        for dz in (1, 2, 3):
            pl.semaphore_signal(
                barrier, inc=1,
                device_id=(my_x, my_y, (my_z + dz) % Z),
                device_id_type=pl.DeviceIdType.MESH,
            )
        pl.semaphore_wait(barrier, 3)

        sends = []
        for dz in (1, 2, 3):
            peer = (my_z + dz) % Z
            rdma = pltpu.make_async_remote_copy(
                src_ref=x_ref.at[:, pl.ds(peer * blk, blk)],
                dst_ref=out_ref.at[pl.ds(my_z * m, m), :],
                send_sem=send_sems.at[dz - 1],
                recv_sem=recv_sems.at[dz - 1],
                device_id=(my_x, my_y, peer),
                device_id_type=pl.DeviceIdType.MESH,
            )
            rdma.start()
            sends.append(rdma)

        local = pltpu.make_async_copy(
            x_ref.at[:, pl.ds(my_z * blk, blk)],
            out_ref.at[pl.ds(my_z * m, m), :],
            local_sem,
        )
        local.start()
        local.wait()

        for dz in (1, 2, 3):
            src_z = (my_z - dz) % Z
            recv = pltpu.make_async_remote_copy(
                src_ref=x_ref.at[:, pl.ds(0, blk)],
                dst_ref=out_ref.at[pl.ds(src_z * m, m), :],
                send_sem=send_sems.at[dz - 1],
                recv_sem=recv_sems.at[dz - 1],
                device_id=(my_x, my_y, src_z),
                device_id_type=pl.DeviceIdType.MESH,
            )
            recv.wait_recv()

        for rdma in sends:
            rdma.wait_send()

    out_shape = jax.ShapeDtypeStruct((Z * m, blk), jnp.bfloat16)
    return pl.pallas_call(
        body,
        out_shape=out_shape,
        in_specs=[pl.BlockSpec(memory_space=pl.ANY)],
        out_specs=pl.BlockSpec(memory_space=pltpu.MemorySpace.VMEM),
        scratch_shapes=[
            pltpu.SemaphoreType.DMA((3,)),
            pltpu.SemaphoreType.DMA((3,)),
            pltpu.SemaphoreType.DMA,
        ],
        compiler_params=pltpu.CompilerParams(collective_id=0),
    )(x)


# device time: 315132 ns/iter; 1.0853x vs baseline; 1.0853x over previous
import jax
import jax.numpy as jnp
from jax import lax
from jax.experimental import pallas as pl
from jax.experimental.pallas import tpu as pltpu

Z = 4
CHUNK = 512


def kernel(x):
    m, n = x.shape
    blk = n // Z
    n_chunks = m // CHUNK

    def body(x_ref, out_ref, xbf_ref, stage_ref, send_sems, recv_sems,
             stage_sems):
        my_x = lax.axis_index("x")
        my_y = lax.axis_index("y")
        my_z = lax.axis_index("z")

        barrier = pltpu.get_barrier_semaphore()
        for dz in (1, 2, 3):
            pl.semaphore_signal(
                barrier, inc=1,
                device_id=(my_x, my_y, (my_z + dz) % Z),
                device_id_type=pl.DeviceIdType.MESH,
            )
        pl.semaphore_wait(barrier, 3)

        def load_chunk(c, col_start, slot):
            cp = pltpu.make_async_copy(
                x_ref.at[pl.ds(c * CHUNK, CHUNK), pl.ds(col_start, blk)],
                stage_ref.at[slot],
                stage_sems.at[slot],
            )
            cp.start()
            return cp

        def convert_block(col_start, store_chunk):
            cp = load_chunk(0, col_start, 0)
            for c in range(n_chunks):
                nxt = load_chunk(c + 1, col_start, (c + 1) % 2) \
                    if c + 1 < n_chunks else None
                cp.wait()
                store_chunk(c, stage_ref[c % 2].astype(jnp.bfloat16))
                cp = nxt

        sends = []
        for dz in (1, 2, 3):
            peer = (my_z + dz) % Z

            def store(c, v, _dz=dz):
                xbf_ref[_dz - 1, pl.ds(c * CHUNK, CHUNK), :] = v

            convert_block(peer * blk, store)
            rdma = pltpu.make_async_remote_copy(
                src_ref=xbf_ref.at[dz - 1],
                dst_ref=out_ref.at[pl.ds(my_z * m, m), :],
                send_sem=send_sems.at[dz - 1],
                recv_sem=recv_sems.at[dz - 1],
                device_id=(my_x, my_y, peer),
                device_id_type=pl.DeviceIdType.MESH,
            )
            rdma.start()
            sends.append(rdma)

        def store_local(c, v):
            out_ref[pl.ds(my_z * m + c * CHUNK, CHUNK), :] = v

        convert_block(my_z * blk, store_local)

        for dz in (1, 2, 3):
            src_z = (my_z - dz) % Z
            recv = pltpu.make_async_remote_copy(
                src_ref=xbf_ref.at[dz - 1],
                dst_ref=out_ref.at[pl.ds(src_z * m, m), :],
                send_sem=send_sems.at[dz - 1],
                recv_sem=recv_sems.at[dz - 1],
                device_id=(my_x, my_y, src_z),
                device_id_type=pl.DeviceIdType.MESH,
            )
            recv.wait_recv()

        for rdma in sends:
            rdma.wait_send()

    out_shape = jax.ShapeDtypeStruct((Z * m, blk), jnp.bfloat16)
    return pl.pallas_call(
        body,
        out_shape=out_shape,
        in_specs=[pl.BlockSpec(memory_space=pl.ANY)],
        out_specs=pl.BlockSpec(memory_space=pltpu.MemorySpace.VMEM),
        scratch_shapes=[
            pltpu.VMEM((3, m, blk), jnp.bfloat16),
            pltpu.VMEM((2, CHUNK, blk), jnp.float32),
            pltpu.SemaphoreType.DMA((3,)),
            pltpu.SemaphoreType.DMA((3,)),
            pltpu.SemaphoreType.DMA((2,)),
        ],
        compiler_params=pltpu.CompilerParams(
            collective_id=0,
            vmem_limit_bytes=63 * 1024 * 1024,
        ),
    )(x)


# device time: 304984 ns/iter; 1.1214x vs baseline; 1.0333x over previous
import jax
import jax.numpy as jnp
from jax import lax
from jax.experimental import pallas as pl
from jax.experimental.pallas import tpu as pltpu

Z = 4
CHUNK = 512


def kernel(x):
    m, n = x.shape
    blk = n // Z
    n_chunks = m // CHUNK

    def body(x_ref, out_ref, xbf_ref, stage_ref, send_sems, recv_sems,
             stage_sems, local_sem):
        my_x = lax.axis_index("x")
        my_y = lax.axis_index("y")
        my_z = lax.axis_index("z")

        barrier = pltpu.get_barrier_semaphore()
        for dz in (1, 2, 3):
            pl.semaphore_signal(
                barrier, inc=1,
                device_id=(my_x, my_y, (my_z + dz) % Z),
                device_id_type=pl.DeviceIdType.MESH,
            )
        pl.semaphore_wait(barrier, 3)

        def load_chunk(c, col_start, slot):
            cp = pltpu.make_async_copy(
                x_ref.at[pl.ds(c * CHUNK, CHUNK), pl.ds(col_start, blk)],
                stage_ref.at[slot],
                stage_sems.at[slot],
            )
            cp.start()
            return cp

        def convert_block(col_start, store_chunk):
            cp = load_chunk(0, col_start, 0)
            for c in range(n_chunks):
                nxt = load_chunk(c + 1, col_start, (c + 1) % 2) \
                    if c + 1 < n_chunks else None
                cp.wait()
                store_chunk(c, stage_ref[c % 2].astype(jnp.bfloat16))
                cp = nxt

        sends = []
        for dz in (1, 2, 3):
            peer = (my_z + dz) % Z

            def store(c, v, _dz=dz):
                xbf_ref[_dz - 1, pl.ds(c * CHUNK, CHUNK), :] = v

            convert_block(peer * blk, store)
            rdma = pltpu.make_async_remote_copy(
                src_ref=xbf_ref.at[dz - 1],
                dst_ref=out_ref.at[pl.ds(my_z * m, m), :],
                send_sem=send_sems.at[dz - 1],
                recv_sem=recv_sems.at[dz - 1],
                device_id=(my_x, my_y, peer),
                device_id_type=pl.DeviceIdType.MESH,
            )
            rdma.start()
            sends.append(rdma)

        def store_local(c, v):
            xbf_ref[3, pl.ds(c * CHUNK, CHUNK), :] = v

        convert_block(my_z * blk, store_local)
        local = pltpu.make_async_copy(
            xbf_ref.at[3],
            out_ref.at[pl.ds(my_z * m, m), :],
            local_sem,
        )
        local.start()

        for dz in (1, 2, 3):
            src_z = (my_z - dz) % Z
            recv = pltpu.make_async_remote_copy(
                src_ref=xbf_ref.at[dz - 1],
                dst_ref=out_ref.at[pl.ds(src_z * m, m), :],
                send_sem=send_sems.at[dz - 1],
                recv_sem=recv_sems.at[dz - 1],
                device_id=(my_x, my_y, src_z),
                device_id_type=pl.DeviceIdType.MESH,
            )
            recv.wait_recv()

        local.wait()
        for rdma in sends:
            rdma.wait_send()

    out_shape = jax.ShapeDtypeStruct((Z * m, blk), jnp.bfloat16)
    return pl.pallas_call(
        body,
        out_shape=out_shape,
        in_specs=[pl.BlockSpec(memory_space=pl.ANY)],
        out_specs=pl.BlockSpec(memory_space=pl.ANY),
        scratch_shapes=[
            pltpu.VMEM((4, m, blk), jnp.bfloat16),
            pltpu.VMEM((2, CHUNK, blk), jnp.float32),
            pltpu.SemaphoreType.DMA((3,)),
            pltpu.SemaphoreType.DMA((3,)),
            pltpu.SemaphoreType.DMA((2,)),
            pltpu.SemaphoreType.DMA,
        ],
        compiler_params=pltpu.CompilerParams(
            collective_id=0,
            vmem_limit_bytes=63 * 1024 * 1024,
        ),
    )(x)
